# baseline (speedup 1.0000x reference)
import functools

import numpy as np
import jax
import jax.numpy as jnp
from jax import lax
from jax.experimental import pallas as pl
from jax.experimental.pallas import tpu as pltpu
from jax.experimental.pallas import tpu_sc as plsc

N_TREES = 100
DEPTH = 9
NPT = 2 ** (DEPTH + 1) - 1
C = 8
M = 2000
NG = 4
CG = C * NG
TG = 4
NGRP = N_TREES // TG
LANES = TG * CG
ROWS = 2 ** (DEPTH + 1)
NW = 32
CHUNK = 128


def _level_perm() -> np.ndarray:
    p = np.zeros(ROWS, np.int64)
    for d in range(DEPTH + 1):
        n = 1 << d
        q = np.arange(n)
        r = np.zeros(n, np.int64)
        for b in range(d):
            r |= ((q >> b) & 1) << (d - 1 - b)
        p[n + q] = (n - 1) + r
    return p


_PERM = _level_perm()

_IL = np.arange(LANES)
_S_NP = ((_IL[:, None] // CG == _IL[None, :] // CG)
         & (_IL[:, None] % NG == _IL[None, :] % NG)).astype(np.float32)


def _emis_table_body(lbt_ref, bt_ref):
    lbt = lbt_ref[...]
    mx = jnp.max(lbt, axis=0, keepdims=True)
    e = jnp.exp(lbt - mx)
    bt_ref[...] = (e / jnp.sum(e, axis=0, keepdims=True)).astype(jnp.bfloat16)


def _emis_table(lbt):
    return pl.pallas_call(
        _emis_table_body,
        out_shape=jax.ShapeDtypeStruct((M, CG), jnp.bfloat16),
    )(lbt)


def _gather_body(table_hbm, idx_hbm, out_hbm, idx_v, rows_v, sem):
    wid = lax.axis_index("s") * 2 + lax.axis_index("c")

    for ch in range(TG):
        @pl.when(wid < NGRP)
        def _():
            pltpu.sync_copy(idx_hbm.at[wid, pl.ds(ch * ROWS, ROWS)], idx_v)
            copies = []
            for k in range(ROWS // CHUNK):
                copies.append(pltpu.async_copy(
                    table_hbm.at[idx_v.at[pl.ds(k * CHUNK, CHUNK)]],
                    rows_v.at[pl.ds(k * CHUNK, CHUNK), :],
                    sem))
            for cp in copies:
                cp.wait()
            pltpu.sync_copy(rows_v, out_hbm.at[wid, ch])


def _gather_rows(table, xpi):
    mesh = plsc.VectorSubcoreMesh(core_axis_name="c", subcore_axis_name="s")
    fn = functools.partial(
        pl.kernel,
        mesh=mesh,
        out_type=jax.ShapeDtypeStruct((NGRP, TG, ROWS, CG), jnp.bfloat16),
        scratch_types=[
            pltpu.VMEM((ROWS,), jnp.int32),
            pltpu.VMEM((ROWS, CG), jnp.bfloat16),
            pltpu.SemaphoreType.DMA,
        ],
        compiler_params=pltpu.CompilerParams(use_tc_tiling_on_sc=False),
    )(_gather_body)
    return fn(table, xpi)


GB = 25


def _upward_body(e_ref, w_ref, p_ref, s_ref, out_ref):
    E = e_ref[...].astype(jnp.float32)
    S = s_ref[...]

    def bcast_dot(a3, mat):
        n = a3.shape[1]
        a2 = a3.reshape(GB * n, LANES)
        return jnp.dot(a2, mat, preferred_element_type=jnp.float32).reshape(
            GB, n, LANES)

    NL = 1 << DEPTH
    b = E[:, NL:2 * NL, :] * p_ref[DEPTH][None, None, :]
    nu = bcast_dot(b, S)
    beta = b / nu
    acc = jnp.sum(jnp.log(nu), axis=1)
    for d in range(DEPTH, 0, -1):
        n = 1 << d
        buv = bcast_dot(beta, w_ref[d])
        u2 = buv[:, :n // 2, :] * buv[:, n // 2:n, :]
        P = E[:, n // 2:n, :] * p_ref[d - 1][None, None, :]
        u = P * P * u2
        nu = bcast_dot(u, S)
        beta = u / nu
        acc = acc + jnp.sum(jnp.log(nu), axis=1)
    out_ref[...] = acc[:, None, :]


def _upward(E3, W, p128):
    return pl.pallas_call(
        _upward_body,
        grid=(NGRP // GB,),
        in_specs=[
            pl.BlockSpec((GB, ROWS, LANES), lambda i: (i, 0, 0)),
            pl.BlockSpec((DEPTH + 1, LANES, LANES), lambda i: (0, 0, 0)),
            pl.BlockSpec((DEPTH + 1, LANES), lambda i: (0, 0)),
            pl.BlockSpec((LANES, LANES), lambda i: (0, 0)),
        ],
        out_specs=pl.BlockSpec((GB, 1, LANES), lambda i: (i, 0, 0)),
        out_shape=jax.ShapeDtypeStruct((NGRP, 1, LANES), jnp.float32),
    )(E3, W, p128, jnp.asarray(_S_NP))


def kernel(x, lambda_A, lambda_B, lambda_Pi):
    A = jax.nn.softmax(lambda_A, axis=0)
    Pi = jax.nn.softmax(lambda_Pi, axis=0)
    ps = [Pi]
    for _ in range(DEPTH):
        ps.append(jnp.einsum('ijg,jg->ig', A, ps[-1]))
    p_all = jnp.stack(ps)
    p128 = jnp.tile(p_all.reshape(DEPTH + 1, CG), (1, TG))
    eye_g = jnp.eye(NG, dtype=jnp.float32)
    ws = []
    for d in range(DEPTH + 1):
        Ad = A / p_all[d][:, None, :]
        w32 = jnp.einsum('ijg,gh->igjh', Ad, eye_g).reshape(CG, CG)
        ws.append(jnp.kron(jnp.eye(TG, dtype=jnp.float32), w32))
    W = jnp.stack(ws)

    lbt = jnp.transpose(lambda_B, (1, 0, 2)).reshape(M, CG)
    x2 = x.reshape(N_TREES, NPT).astype(jnp.int32)
    xp = x2[:, _PERM]
    xpi = xp.reshape(NGRP, TG, ROWS).transpose(0, 2, 1).reshape(NGRP, TG * ROWS)

    table = _emis_table(lbt)
    E4 = _gather_rows(table, xpi)
    E3 = E4.reshape(NGRP, ROWS, LANES)
    out128 = _upward(E3, W, p128)
    return out128.reshape(NGRP, TG, CG)[:, :, :NG].reshape(N_TREES, NG)


# speedup vs baseline: 1.3821x; 1.3821x over previous
import functools

import numpy as np
import jax
import jax.numpy as jnp
from jax import lax
from jax.experimental import pallas as pl
from jax.experimental.pallas import tpu as pltpu
from jax.experimental.pallas import tpu_sc as plsc

N_TREES = 100
DEPTH = 9
NPT = 2 ** (DEPTH + 1) - 1
C = 8
M = 2000
NG = 4
CG = C * NG
TG = 4
NGRP = N_TREES // TG
LANES = TG * CG
ROWS = 2 ** (DEPTH + 1)
NW = 32
CHUNK = 128


def _level_perm() -> np.ndarray:
    p = np.zeros(ROWS, np.int64)
    for d in range(DEPTH + 1):
        n = 1 << d
        q = np.arange(n)
        r = np.zeros(n, np.int64)
        for b in range(d):
            r |= ((q >> b) & 1) << (d - 1 - b)
        p[n + q] = (n - 1) + r
    return p


_PERM = _level_perm()

_IL = np.arange(LANES)
_S_NP = ((_IL[:, None] // CG == _IL[None, :] // CG)
         & (_IL[:, None] % NG == _IL[None, :] % NG)).astype(np.float32)


def _emis_table_body(lbt_ref, bt_ref):
    lbt = lbt_ref[...]
    mx = jnp.max(lbt, axis=0, keepdims=True)
    e = jnp.exp(lbt - mx)
    bt_ref[...] = e / jnp.sum(e, axis=0, keepdims=True)


def _emis_table(lbt):
    return pl.pallas_call(
        _emis_table_body,
        out_shape=jax.ShapeDtypeStruct((M, CG), jnp.float32),
    )(lbt)


def _gather_body(table_hbm, idx_hbm, out_hbm,
                 idx_v, rows0, rows1, gsem, wsem0, wsem1):
    wid = lax.axis_index("s") * 2 + lax.axis_index("c")
    rows_b = (rows0, rows1)
    wsem_b = (wsem0, wsem1)

    for ch in range(TG):
        b = ch & 1

        @pl.when(wid < NGRP)
        def _(b=b, ch=ch):
            if ch >= 2:
                pltpu.make_async_copy(
                    rows_b[b], out_hbm.at[0, 0], wsem_b[b]).wait()
            pltpu.sync_copy(idx_hbm.at[wid, pl.ds(ch * ROWS, ROWS)], idx_v)
            copies = []
            for k in range(ROWS // CHUNK):
                copies.append(pltpu.async_copy(
                    table_hbm.at[idx_v.at[pl.ds(k * CHUNK, CHUNK)]],
                    rows_b[b].at[pl.ds(k * CHUNK, CHUNK), :],
                    gsem))
            for cp in copies:
                cp.wait()
            pltpu.async_copy(rows_b[b], out_hbm.at[wid, ch], wsem_b[b])

    @pl.when(wid < NGRP)
    def _():
        for b in range(2):
            pltpu.make_async_copy(rows_b[b], out_hbm.at[0, 0], wsem_b[b]).wait()


def _gather_rows(table, xpi):
    mesh = plsc.VectorSubcoreMesh(core_axis_name="c", subcore_axis_name="s")
    fn = functools.partial(
        pl.kernel,
        mesh=mesh,
        out_type=jax.ShapeDtypeStruct((NGRP, TG, ROWS, CG), jnp.float32),
        scratch_types=[
            pltpu.VMEM((ROWS,), jnp.int32),
            pltpu.VMEM((ROWS, CG), jnp.float32),
            pltpu.VMEM((ROWS, CG), jnp.float32),
            pltpu.SemaphoreType.DMA,
            pltpu.SemaphoreType.DMA,
            pltpu.SemaphoreType.DMA,
        ],
        compiler_params=pltpu.CompilerParams(use_tc_tiling_on_sc=False),
    )(_gather_body)
    return fn(table, xpi)


GB = 25


def _upward_body(e_ref, w_ref, p_ref, s_ref, out_ref):
    E = e_ref[...]
    S = s_ref[...]

    def bcast_dot(a3, mat):
        n = a3.shape[1]
        a2 = a3.reshape(GB * n, LANES)
        return jnp.dot(a2, mat, preferred_element_type=jnp.float32).reshape(
            GB, n, LANES)

    NL = 1 << DEPTH
    b = E[:, NL:2 * NL, :] * p_ref[DEPTH][None, None, :]
    nu = bcast_dot(b, S)
    beta = b / nu
    acc = jnp.sum(jnp.log(nu), axis=1)
    for d in range(DEPTH, 0, -1):
        n = 1 << d
        buv = bcast_dot(beta, w_ref[d])
        u2 = buv[:, :n // 2, :] * buv[:, n // 2:n, :]
        P = E[:, n // 2:n, :] * p_ref[d - 1][None, None, :]
        u = P * P * u2
        nu = bcast_dot(u, S)
        beta = u / nu
        acc = acc + jnp.sum(jnp.log(nu), axis=1)
    out_ref[...] = acc[:, None, :]


def _upward(E3, W, p128):
    return pl.pallas_call(
        _upward_body,
        grid=(NGRP // GB,),
        in_specs=[
            pl.BlockSpec((GB, ROWS, LANES), lambda i: (i, 0, 0)),
            pl.BlockSpec((DEPTH + 1, LANES, LANES), lambda i: (0, 0, 0)),
            pl.BlockSpec((DEPTH + 1, LANES), lambda i: (0, 0)),
            pl.BlockSpec((LANES, LANES), lambda i: (0, 0)),
        ],
        out_specs=pl.BlockSpec((GB, 1, LANES), lambda i: (i, 0, 0)),
        out_shape=jax.ShapeDtypeStruct((NGRP, 1, LANES), jnp.float32),
    )(E3, W, p128, jnp.asarray(_S_NP))


def kernel(x, lambda_A, lambda_B, lambda_Pi):
    A = jax.nn.softmax(lambda_A, axis=0)
    Pi = jax.nn.softmax(lambda_Pi, axis=0)
    ps = [Pi]
    for _ in range(DEPTH):
        ps.append(jnp.einsum('ijg,jg->ig', A, ps[-1]))
    p_all = jnp.stack(ps)
    p128 = jnp.tile(p_all.reshape(DEPTH + 1, CG), (1, TG))
    eye_g = jnp.eye(NG, dtype=jnp.float32)
    ws = []
    for d in range(DEPTH + 1):
        Ad = A / p_all[d][:, None, :]
        w32 = jnp.einsum('ijg,gh->igjh', Ad, eye_g).reshape(CG, CG)
        ws.append(jnp.kron(jnp.eye(TG, dtype=jnp.float32), w32))
    W = jnp.stack(ws)

    lbt = jnp.transpose(lambda_B, (1, 0, 2)).reshape(M, CG)
    x2 = x.reshape(N_TREES, NPT).astype(jnp.int32)
    xp = x2[:, _PERM]
    xpi = xp.reshape(NGRP, TG, ROWS).transpose(0, 2, 1).reshape(NGRP, TG * ROWS)

    table = _emis_table(lbt)
    E4 = _gather_rows(table, xpi)
    E3 = E4.reshape(NGRP, ROWS, LANES)
    out128 = _upward(E3, W, p128)
    return out128.reshape(NGRP, TG, CG)[:, :, :NG].reshape(N_TREES, NG)


# speedup vs baseline: 1.4198x; 1.0273x over previous
import functools

import numpy as np
import jax
import jax.numpy as jnp
from jax import lax
from jax.experimental import pallas as pl
from jax.experimental.pallas import tpu as pltpu
from jax.experimental.pallas import tpu_sc as plsc

N_TREES = 100
DEPTH = 9
NPT = 2 ** (DEPTH + 1) - 1
C = 8
M = 2000
NG = 4
CG = C * NG
TG = 4
NGRP = N_TREES // TG
LANES = TG * CG
ROWS = 2 ** (DEPTH + 1)
NW = 32
CHUNK = 128


def _level_perm() -> np.ndarray:
    p = np.zeros(ROWS, np.int64)
    for d in range(DEPTH + 1):
        n = 1 << d
        q = np.arange(n)
        r = np.zeros(n, np.int64)
        for b in range(d):
            r |= ((q >> b) & 1) << (d - 1 - b)
        p[n + q] = (n - 1) + r
    return p


_PERM = _level_perm()

_IL = np.arange(LANES)
_S_NP = ((_IL[:, None] // CG == _IL[None, :] // CG)
         & (_IL[:, None] % NG == _IL[None, :] % NG)).astype(np.float32)


def _emis_table_body(lbt_ref, bt_ref):
    lbt = lbt_ref[...]
    mx = jnp.max(lbt, axis=0, keepdims=True)
    e = jnp.exp(lbt - mx)
    bt_ref[...] = e / jnp.sum(e, axis=0, keepdims=True)


def _emis_table(lbt):
    return pl.pallas_call(
        _emis_table_body,
        out_shape=jax.ShapeDtypeStruct((M, CG), jnp.float32),
    )(lbt)


def _gather_body(table_hbm, idx_hbm, out_hbm, idx_v, rows_v, sem):
    wid = lax.axis_index("s") * 2 + lax.axis_index("c")

    for ch in range(TG):
        @pl.when(wid < NGRP)
        def _():
            pltpu.sync_copy(idx_hbm.at[wid, pl.ds(ch * ROWS, ROWS)], idx_v)
            copies = []
            for k in range(ROWS // CHUNK):
                copies.append(pltpu.async_copy(
                    table_hbm.at[idx_v.at[pl.ds(k * CHUNK, CHUNK)]],
                    rows_v.at[pl.ds(k * CHUNK, CHUNK), :],
                    sem))
            for cp in copies:
                cp.wait()
            pltpu.sync_copy(rows_v, out_hbm.at[wid, ch])


def _gather_rows(table, xpi):
    mesh = plsc.VectorSubcoreMesh(core_axis_name="c", subcore_axis_name="s")
    fn = functools.partial(
        pl.kernel,
        mesh=mesh,
        out_type=jax.ShapeDtypeStruct((NGRP, TG, ROWS, CG), jnp.float32),
        scratch_types=[
            pltpu.VMEM((ROWS,), jnp.int32),
            pltpu.VMEM((ROWS, CG), jnp.float32),
            pltpu.SemaphoreType.DMA,
        ],
        compiler_params=pltpu.CompilerParams(use_tc_tiling_on_sc=False),
    )(_gather_body)
    return fn(table, xpi)


GB = 25


_LN2 = 0.6931471805599453


def _upward_body(e_ref, w_ref, p_ref, s_ref, out_ref):
    E = e_ref[...]
    S = s_ref[...]

    def bcast_dot(a3, mat):
        n = a3.shape[1]
        a2 = a3.reshape(GB * n, LANES)
        return jnp.dot(a2, mat, preferred_element_type=jnp.float32).reshape(
            GB, n, LANES)

    def pow2_norm(u3):
        nu = bcast_dot(u3, S)
        bits = lax.bitcast_convert_type(nu, jnp.int32)
        k = jnp.right_shift(bits, 23) - 127
        scale = lax.bitcast_convert_type(
            jnp.left_shift(127 - k, 23), jnp.float32)
        return u3 * scale, k, nu * scale

    NL = 1 << DEPTH
    b = E[:, NL:2 * NL, :] * p_ref[DEPTH][None, None, :]
    beta, k3, sig = pow2_norm(b)
    acc_k = jnp.sum(k3, axis=1)
    for d in range(DEPTH, 0, -1):
        n = 1 << d
        buv = bcast_dot(beta, w_ref[d])
        u2 = buv[:, :n // 2, :] * buv[:, n // 2:n, :]
        P = E[:, n // 2:n, :] * p_ref[d - 1][None, None, :]
        u = P * P * u2
        beta, k3, sig = pow2_norm(u)
        acc_k = acc_k + jnp.sum(k3, axis=1)
    out = acc_k.astype(jnp.float32) * _LN2 + jnp.log(sig[:, 0, :])
    out_ref[...] = out[:, None, :]


def _upward(E3, W, p128):
    return pl.pallas_call(
        _upward_body,
        grid=(NGRP // GB,),
        in_specs=[
            pl.BlockSpec((GB, ROWS, LANES), lambda i: (i, 0, 0)),
            pl.BlockSpec((DEPTH + 1, LANES, LANES), lambda i: (0, 0, 0)),
            pl.BlockSpec((DEPTH + 1, LANES), lambda i: (0, 0)),
            pl.BlockSpec((LANES, LANES), lambda i: (0, 0)),
        ],
        out_specs=pl.BlockSpec((GB, 1, LANES), lambda i: (i, 0, 0)),
        out_shape=jax.ShapeDtypeStruct((NGRP, 1, LANES), jnp.float32),
    )(E3, W, p128, jnp.asarray(_S_NP))


def kernel(x, lambda_A, lambda_B, lambda_Pi):
    A = jax.nn.softmax(lambda_A, axis=0)
    Pi = jax.nn.softmax(lambda_Pi, axis=0)
    ps = [Pi]
    for _ in range(DEPTH):
        ps.append(jnp.einsum('ijg,jg->ig', A, ps[-1]))
    p_all = jnp.stack(ps)
    p128 = jnp.tile(p_all.reshape(DEPTH + 1, CG), (1, TG))
    eye_g = jnp.eye(NG, dtype=jnp.float32)
    ws = []
    for d in range(DEPTH + 1):
        Ad = A / p_all[d][:, None, :]
        w32 = jnp.einsum('ijg,gh->igjh', Ad, eye_g).reshape(CG, CG)
        ws.append(jnp.kron(jnp.eye(TG, dtype=jnp.float32), w32))
    W = jnp.stack(ws)

    lbt = jnp.transpose(lambda_B, (1, 0, 2)).reshape(M, CG)
    x2 = x.reshape(N_TREES, NPT).astype(jnp.int32)
    xp = x2[:, _PERM]
    xpi = xp.reshape(NGRP, TG, ROWS).transpose(0, 2, 1).reshape(NGRP, TG * ROWS)

    table = _emis_table(lbt)
    E4 = _gather_rows(table, xpi)
    E3 = E4.reshape(NGRP, ROWS, LANES)
    out128 = _upward(E3, W, p128)
    return out128.reshape(NGRP, TG, CG)[:, :, :NG].reshape(N_TREES, NG)
